# Initial kernel scaffold; baseline (speedup 1.0000x reference)
import jax
import jax.numpy as jnp
from jax.experimental import pallas as pl


def kernel(x, hash_W, bases, conv_W, depth):
    raise NotImplementedError("write your pallas kernel here")


# speedup vs baseline: 5.2252x; 5.2252x over previous
import functools

import jax
import jax.numpy as jnp
from jax import lax
from jax.experimental import pallas as pl
from jax.experimental.pallas import tpu as pltpu

B = 64
U = 16
D = 256
K = 32
C = 32
HW = 1024
F = C * HW
DEPTH = 4
UK = U * K
UD = U * D
KBLK = 4096
NKB = F // KBLK


def _hash_kernel(x_ref, hw_ref, b2_ref, b2t_ref, h_ref, p2_ref):
    k = pl.program_id(0)

    @pl.when(k == 0)
    def _init():
        h_ref[...] = jnp.zeros_like(h_ref)

    h_ref[...] += jnp.dot(x_ref[...].astype(jnp.bfloat16), hw_ref[...],
                          preferred_element_type=jnp.float32)

    @pl.when(k == NKB - 1)
    def _fin():
        h = h_ref[...]
        coef = jnp.dot(h.astype(jnp.bfloat16), b2_ref[...],
                       preferred_element_type=jnp.float32)
        cbf = coef.astype(jnp.bfloat16)
        for u in range(U):
            p2_ref[:, u * D:(u + 1) * D] = jnp.dot(
                cbf[:, u * K:(u + 1) * K], b2t_ref[u * K:(u + 1) * K, :],
                preferred_element_type=jnp.float32)


def _hash_call(xf, hw_bf, B2bf, B2Tbf):
    return pl.pallas_call(
        _hash_kernel,
        grid=(NKB,),
        in_specs=[
            pl.BlockSpec((B, KBLK), lambda k: (0, k)),
            pl.BlockSpec((KBLK, D), lambda k: (k, 0)),
            pl.BlockSpec((D, UK), lambda k: (0, 0)),
            pl.BlockSpec((UK, D), lambda k: (0, 0)),
        ],
        out_specs=[
            pl.BlockSpec((B, D), lambda k: (0, 0)),
            pl.BlockSpec((B, UD), lambda k: (0, 0)),
        ],
        out_shape=[
            jax.ShapeDtypeStruct((B, D), jnp.float32),
            jax.ShapeDtypeStruct((B, UD), jnp.float32),
        ],
    )(xf, hw_bf, B2bf, B2Tbf)


def _route_kernel(d_const, depth_ref, avail_ref, p2_ref, h_ref,
                  outp_ref, act_ref,
                  route_ref, idx_ref, avail_o_ref, out_o_ref, act_o_ref):
    step_on = d_const < depth_ref[0]
    p2 = p2_ref[...]
    mags2 = jnp.stack(
        [jnp.sum(p2[:, u * D:(u + 1) * D] ** 2, axis=1) for u in range(U)],
        axis=1)
    avail = avail_ref[...]
    masked = jnp.where(avail > 0.5, mags2, -jnp.inf)
    m = jnp.max(masked, axis=1, keepdims=True)
    iota = lax.broadcasted_iota(jnp.int32, (B, U), 1)
    idx = jnp.min(jnp.where(masked == m, iota, U + 1), axis=1)
    onehot = (iota == idx[:, None]).astype(jnp.float32)
    proj = jnp.zeros((B, D), jnp.float32)
    for u in range(U):
        proj = proj + p2[:, u * D:(u + 1) * D] * onehot[:, u][:, None]
    residual = h_ref[...] - proj
    act = act_ref[...][:, 0]
    sonf = jnp.where(step_on, 1.0, 0.0)
    out_o_ref[...] = outp_ref[...] + residual * (act * sonf)[:, None]
    rnorm2 = jnp.sum(residual * residual, axis=1)
    live = (rnorm2 > 1e-10).astype(jnp.float32)
    act_o_ref[...] = jnp.where(step_on, act * live, act)[:, None]
    avail_o_ref[...] = jnp.where(step_on, avail * (1.0 - onehot), avail)
    route_ref[...] = jnp.where(step_on, idx, 0)[:, None]
    idx_ref[...] = idx[:, None]


def _route_call(d, depth_arr, avail, p2, h, outp, act):
    return pl.pallas_call(
        functools.partial(_route_kernel, d),
        in_specs=[
            pl.BlockSpec(memory_space=pltpu.SMEM),
            pl.BlockSpec((B, U), lambda: (0, 0)),
            pl.BlockSpec((B, UD), lambda: (0, 0)),
            pl.BlockSpec((B, D), lambda: (0, 0)),
            pl.BlockSpec((B, D), lambda: (0, 0)),
            pl.BlockSpec((B, 1), lambda: (0, 0)),
        ],
        out_specs=[
            pl.BlockSpec((B, 1), lambda: (0, 0)),
            pl.BlockSpec((B, 1), lambda: (0, 0)),
            pl.BlockSpec((B, U), lambda: (0, 0)),
            pl.BlockSpec((B, D), lambda: (0, 0)),
            pl.BlockSpec((B, 1), lambda: (0, 0)),
        ],
        out_shape=[
            jax.ShapeDtypeStruct((B, 1), jnp.int32),
            jax.ShapeDtypeStruct((B, 1), jnp.int32),
            jax.ShapeDtypeStruct((B, U), jnp.float32),
            jax.ShapeDtypeStruct((B, D), jnp.float32),
            jax.ShapeDtypeStruct((B, 1), jnp.float32),
        ],
    )(depth_arr, avail, p2, h, outp, act)


def _conv_kernel(idx_ref, x_ref, w_ref, o_ref):
    xb = x_ref[0].astype(jnp.bfloat16)
    wpos = lax.broadcasted_iota(jnp.int32, (C, HW), 1) % 32
    mnr = (wpos != 31).astype(jnp.bfloat16)
    mnl = (wpos != 0).astype(jnp.bfloat16)
    acc = jnp.zeros((C, HW), jnp.float32)
    for t in range(9):
        ky, kx = t // 3, t % 3
        dy, dx = ky - 1, kx - 1
        sh = 32 * dy + dx
        if sh > 0:
            s = jnp.concatenate(
                [xb[:, sh:], jnp.zeros((C, sh), jnp.bfloat16)], axis=1)
        elif sh < 0:
            s = jnp.concatenate(
                [jnp.zeros((C, -sh), jnp.bfloat16), xb[:, :sh]], axis=1)
        else:
            s = xb
        if dx == 1:
            s = s * mnr
        elif dx == -1:
            s = s * mnl
        acc += jnp.dot(w_ref[0, t], s, preferred_element_type=jnp.float32)
    o_ref[0] = jnp.maximum(acc, 0.0)


def _conv_call(idx, x3, CWr_bf):
    return pl.pallas_call(
        _conv_kernel,
        grid_spec=pltpu.PrefetchScalarGridSpec(
            num_scalar_prefetch=1,
            grid=(B,),
            in_specs=[
                pl.BlockSpec((1, C, HW), lambda b, idx_ref: (b, 0, 0)),
                pl.BlockSpec((1, 9, C, C),
                             lambda b, idx_ref: (idx_ref[b], 0, 0, 0)),
            ],
            out_specs=pl.BlockSpec((1, C, HW), lambda b, idx_ref: (b, 0, 0)),
        ),
        out_shape=jax.ShapeDtypeStruct((B, C, HW), jnp.float32),
    )(idx, x3, CWr_bf)


def kernel(x, hash_W, bases, conv_W, depth):
    depth_arr = jnp.asarray(depth, jnp.int32).reshape(1)
    xf = x.reshape(B, F)
    hw_bf = hash_W.astype(jnp.bfloat16)
    B2bf = bases.transpose(1, 0, 2).reshape(D, UK).astype(jnp.bfloat16)
    B2Tbf = bases.transpose(0, 2, 1).reshape(UK, D).astype(jnp.bfloat16)
    CWr_bf = conv_W.transpose(0, 3, 4, 1, 2).reshape(U, 9, C, C).astype(
        jnp.bfloat16)

    outp = jnp.zeros((B, D), jnp.float32)
    avail = jnp.ones((B, U), jnp.float32)
    act = jnp.ones((B, 1), jnp.float32)
    routes = []
    xcur = xf
    for d in range(DEPTH):
        h, p2 = _hash_call(xcur, hw_bf, B2bf, B2Tbf)
        route_d, idx2, avail, outp, act = _route_call(
            d, depth_arr, avail, p2, h, outp, act)
        routes.append(route_d[:, 0])
        if d < DEPTH - 1:
            x3 = xcur.reshape(B, C, HW)
            xcur = _conv_call(idx2[:, 0], x3, CWr_bf).reshape(B, F)
    return outp, jnp.stack(routes, axis=1)


# speedup vs baseline: 6.1884x; 1.1843x over previous
import functools

import jax
import jax.numpy as jnp
from jax import lax
from jax.experimental import pallas as pl
from jax.experimental.pallas import tpu as pltpu

B = 64
U = 16
D = 256
K = 32
C = 32
HW = 1024
F = C * HW
DEPTH = 4
UK = U * K
UD = U * D
CBLK = 8
NKB = C // CBLK


def _hash_route_kernel(d_const, depth_ref, x_ref, hw_ref, b2_ref, b2t_ref,
                       avail_ref, outp_ref, act_ref,
                       h_ref, route_ref, idx_ref, avail_o_ref, out_o_ref,
                       act_o_ref):
    kg = pl.program_id(0)

    @pl.when(kg == 0)
    def _init():
        h_ref[...] = jnp.zeros_like(h_ref)

    acc = jnp.zeros((B, D), jnp.float32)
    for j in range(CBLK):
        xc = x_ref[:, j, :].astype(jnp.bfloat16)
        acc = acc + jnp.dot(xc, hw_ref[j * HW:(j + 1) * HW, :],
                            preferred_element_type=jnp.float32)
    h_ref[...] += acc

    @pl.when(kg == NKB - 1)
    def _fin():
        h = h_ref[...]
        coef = jnp.dot(h.astype(jnp.bfloat16), b2_ref[...],
                       preferred_element_type=jnp.float32)
        cbf = coef.astype(jnp.bfloat16)
        projs = []
        m2 = []
        for u in range(U):
            pu = jnp.dot(cbf[:, u * K:(u + 1) * K],
                         b2t_ref[u * K:(u + 1) * K, :],
                         preferred_element_type=jnp.float32)
            projs.append(pu)
            m2.append(jnp.sum(pu * pu, axis=1))
        mags2 = jnp.stack(m2, axis=1)
        step_on = d_const < depth_ref[0]
        avail = avail_ref[...]
        masked = jnp.where(avail > 0.5, mags2, -jnp.inf)
        m = jnp.max(masked, axis=1, keepdims=True)
        iota = lax.broadcasted_iota(jnp.int32, (B, U), 1)
        idx = jnp.min(jnp.where(masked == m, iota, U + 1), axis=1)
        onehot = (iota == idx[:, None]).astype(jnp.float32)
        proj = jnp.zeros((B, D), jnp.float32)
        for u in range(U):
            proj = proj + projs[u] * onehot[:, u][:, None]
        residual = h - proj
        act = act_ref[...][:, 0]
        sonf = jnp.where(step_on, 1.0, 0.0)
        out_o_ref[...] = outp_ref[...] + residual * (act * sonf)[:, None]
        rnorm2 = jnp.sum(residual * residual, axis=1)
        live = (rnorm2 > 1e-10).astype(jnp.float32)
        act_o_ref[...] = jnp.where(step_on, act * live, act)[:, None]
        avail_o_ref[...] = jnp.where(step_on, avail * (1.0 - onehot), avail)
        route_ref[...] = jnp.where(step_on, idx, 0)[:, None]
        idx_ref[...] = idx[:, None]


def _hash_route_call(d, depth_arr, x3, hw_bf, B2bf, B2Tbf, avail, outp, act):
    full = lambda k: (0, 0)
    return pl.pallas_call(
        functools.partial(_hash_route_kernel, d),
        grid=(NKB,),
        in_specs=[
            pl.BlockSpec(memory_space=pltpu.SMEM),
            pl.BlockSpec((B, CBLK, HW), lambda k: (0, k, 0)),
            pl.BlockSpec((CBLK * HW, D), lambda k: (k, 0)),
            pl.BlockSpec((D, UK), full),
            pl.BlockSpec((UK, D), full),
            pl.BlockSpec((B, U), full),
            pl.BlockSpec((B, D), full),
            pl.BlockSpec((B, 1), full),
        ],
        out_specs=[
            pl.BlockSpec((B, D), full),
            pl.BlockSpec((B, 1), full),
            pl.BlockSpec((B, 1), full),
            pl.BlockSpec((B, U), full),
            pl.BlockSpec((B, D), full),
            pl.BlockSpec((B, 1), full),
        ],
        out_shape=[
            jax.ShapeDtypeStruct((B, D), jnp.float32),
            jax.ShapeDtypeStruct((B, 1), jnp.int32),
            jax.ShapeDtypeStruct((B, 1), jnp.int32),
            jax.ShapeDtypeStruct((B, U), jnp.float32),
            jax.ShapeDtypeStruct((B, D), jnp.float32),
            jax.ShapeDtypeStruct((B, 1), jnp.float32),
        ],
    )(depth_arr, x3, hw_bf, B2bf, B2Tbf, avail, outp, act)


def _conv_kernel(idx_ref, x_ref, w_ref, o_ref):
    xb = x_ref[0].astype(jnp.bfloat16)
    wpos = lax.broadcasted_iota(jnp.int32, (C, HW), 1) % 32
    mnr = (wpos != 31).astype(jnp.bfloat16)
    mnl = (wpos != 0).astype(jnp.bfloat16)
    acc = jnp.zeros((C, HW), jnp.float32)
    for t in range(9):
        ky, kx = t // 3, t % 3
        dy, dx = ky - 1, kx - 1
        sh = 32 * dy + dx
        if sh > 0:
            s = jnp.concatenate(
                [xb[:, sh:], jnp.zeros((C, sh), jnp.bfloat16)], axis=1)
        elif sh < 0:
            s = jnp.concatenate(
                [jnp.zeros((C, -sh), jnp.bfloat16), xb[:, :sh]], axis=1)
        else:
            s = xb
        if dx == 1:
            s = s * mnr
        elif dx == -1:
            s = s * mnl
        acc += jnp.dot(w_ref[0, t], s, preferred_element_type=jnp.float32)
    o_ref[0] = jnp.maximum(acc, 0.0)


def _conv_call(idx, x3, CWr_bf):
    return pl.pallas_call(
        _conv_kernel,
        grid_spec=pltpu.PrefetchScalarGridSpec(
            num_scalar_prefetch=1,
            grid=(B,),
            in_specs=[
                pl.BlockSpec((1, C, HW), lambda b, idx_ref: (b, 0, 0)),
                pl.BlockSpec((1, 9, C, C),
                             lambda b, idx_ref: (idx_ref[b], 0, 0, 0)),
            ],
            out_specs=pl.BlockSpec((1, C, HW), lambda b, idx_ref: (b, 0, 0)),
        ),
        out_shape=jax.ShapeDtypeStruct((B, C, HW), jnp.float32),
    )(idx, x3, CWr_bf)


def kernel(x, hash_W, bases, conv_W, depth):
    depth_arr = jnp.asarray(depth, jnp.int32).reshape(1)
    x3 = x.reshape(B, C, HW)
    hw_bf = hash_W.astype(jnp.bfloat16)
    B2bf = bases.transpose(1, 0, 2).reshape(D, UK).astype(jnp.bfloat16)
    B2Tbf = bases.transpose(0, 2, 1).reshape(UK, D).astype(jnp.bfloat16)
    CWr_bf = conv_W.transpose(0, 3, 4, 1, 2).reshape(U, 9, C, C).astype(
        jnp.bfloat16)

    outp = jnp.zeros((B, D), jnp.float32)
    avail = jnp.ones((B, U), jnp.float32)
    act = jnp.ones((B, 1), jnp.float32)
    routes = []
    xcur = x3
    for d in range(DEPTH):
        h, route_d, idx2, avail, outp, act = _hash_route_call(
            d, depth_arr, xcur, hw_bf, B2bf, B2Tbf, avail, outp, act)
        routes.append(route_d[:, 0])
        if d < DEPTH - 1:
            xcur = _conv_call(idx2[:, 0], xcur, CWr_bf)
    return outp, jnp.stack(routes, axis=1)


# speedup vs baseline: 10.3338x; 1.6699x over previous
import functools

import jax
import jax.numpy as jnp
from jax import lax
from jax.experimental import pallas as pl
from jax.experimental.pallas import tpu as pltpu

B = 64
U = 16
D = 256
K = 32
C = 32
HW = 1024
F = C * HW
DEPTH = 4
UK = U * K
UD = U * D
CBLK = 8
NKB = C // CBLK


def _hash_route_kernel(d_const, cast_hw, *refs):
    if cast_hw:
        (depth_ref, x_ref, hw_ref, b2_ref, b2t_ref,
         avail_ref, outp_ref, act_ref,
         h_ref, route_ref, idx_ref, avail_o_ref, out_o_ref, act_o_ref,
         hwbf_ref) = refs
    else:
        (depth_ref, x_ref, hw_ref, b2_ref, b2t_ref,
         avail_ref, outp_ref, act_ref,
         h_ref, route_ref, idx_ref, avail_o_ref, out_o_ref,
         act_o_ref) = refs
    kg = pl.program_id(0)

    @pl.when(kg == 0)
    def _init():
        h_ref[...] = jnp.zeros_like(h_ref)

    if cast_hw:
        hwc = hw_ref[...].astype(jnp.bfloat16)
        hwbf_ref[...] = hwc
    else:
        hwc = hw_ref[...]
    acc = jnp.zeros((B, D), jnp.float32)
    for j in range(CBLK):
        xc = x_ref[:, j, :].astype(jnp.bfloat16)
        acc = acc + jnp.dot(xc, hwc[j * HW:(j + 1) * HW, :],
                            preferred_element_type=jnp.float32)
    h_ref[...] += acc

    @pl.when(kg == NKB - 1)
    def _fin():
        h = h_ref[...]
        coef = jnp.dot(h.astype(jnp.bfloat16), b2_ref[...],
                       preferred_element_type=jnp.float32)
        cbf = coef.astype(jnp.bfloat16)
        projs = []
        m2 = []
        for u in range(U):
            pu = jnp.dot(cbf[:, u * K:(u + 1) * K],
                         b2t_ref[u * K:(u + 1) * K, :],
                         preferred_element_type=jnp.float32)
            projs.append(pu)
            m2.append(jnp.sum(pu * pu, axis=1))
        mags2 = jnp.stack(m2, axis=1)
        step_on = d_const < depth_ref[0]
        avail = avail_ref[...]
        masked = jnp.where(avail > 0.5, mags2, -jnp.inf)
        m = jnp.max(masked, axis=1, keepdims=True)
        iota = lax.broadcasted_iota(jnp.int32, (B, U), 1)
        idx = jnp.min(jnp.where(masked == m, iota, U + 1), axis=1)
        onehot = (iota == idx[:, None]).astype(jnp.float32)
        proj = jnp.zeros((B, D), jnp.float32)
        for u in range(U):
            proj = proj + projs[u] * onehot[:, u][:, None]
        residual = h - proj
        act = act_ref[...][:, 0]
        sonf = jnp.where(step_on, 1.0, 0.0)
        out_o_ref[...] = outp_ref[...] + residual * (act * sonf)[:, None]
        rnorm2 = jnp.sum(residual * residual, axis=1)
        live = (rnorm2 > 1e-10).astype(jnp.float32)
        act_o_ref[...] = jnp.where(step_on, act * live, act)[:, None]
        avail_o_ref[...] = jnp.where(step_on, avail * (1.0 - onehot), avail)
        route_ref[...] = jnp.where(step_on, idx, 0)[:, None]
        idx_ref[...] = idx[:, None]


def _hash_route_call(d, depth_arr, x3, hw, B2bf, B2Tbf, avail, outp, act,
                     cast_hw=False):
    full = lambda k: (0, 0)
    out_specs = [
        pl.BlockSpec((B, D), full),
        pl.BlockSpec((B, 1), full),
        pl.BlockSpec((B, 1), full),
        pl.BlockSpec((B, U), full),
        pl.BlockSpec((B, D), full),
        pl.BlockSpec((B, 1), full),
    ]
    out_shape = [
        jax.ShapeDtypeStruct((B, D), jnp.float32),
        jax.ShapeDtypeStruct((B, 1), jnp.int32),
        jax.ShapeDtypeStruct((B, 1), jnp.int32),
        jax.ShapeDtypeStruct((B, U), jnp.float32),
        jax.ShapeDtypeStruct((B, D), jnp.float32),
        jax.ShapeDtypeStruct((B, 1), jnp.float32),
    ]
    if cast_hw:
        out_specs.append(pl.BlockSpec((CBLK * HW, D), lambda k: (k, 0)))
        out_shape.append(jax.ShapeDtypeStruct((F, D), jnp.bfloat16))
    return pl.pallas_call(
        functools.partial(_hash_route_kernel, d, cast_hw),
        grid=(NKB,),
        in_specs=[
            pl.BlockSpec(memory_space=pltpu.SMEM),
            pl.BlockSpec((B, CBLK, HW), lambda k: (0, k, 0)),
            pl.BlockSpec((CBLK * HW, D), lambda k: (k, 0)),
            pl.BlockSpec((D, UK), full),
            pl.BlockSpec((UK, D), full),
            pl.BlockSpec((B, U), full),
            pl.BlockSpec((B, D), full),
            pl.BlockSpec((B, 1), full),
        ],
        out_specs=out_specs,
        out_shape=out_shape,
    )(depth_arr, x3, hw, B2bf, B2Tbf, avail, outp, act)


GB = 8


def _conv_kernel(idx_ref, x_ref, w_ref, o_ref):
    g = pl.program_id(0)
    wpos = lax.broadcasted_iota(jnp.int32, (C, HW), 1) % 32
    mnr = (wpos != 31).astype(jnp.bfloat16)
    mnl = (wpos != 0).astype(jnp.bfloat16)
    for e in range(GB):
        ib = idx_ref[g * GB + e]
        xb = x_ref[e].astype(jnp.bfloat16)
        acc = jnp.zeros((C, HW), jnp.float32)
        for t in range(9):
            ky, kx = t // 3, t % 3
            dy, dx = ky - 1, kx - 1
            sh = 32 * dy + dx
            if sh > 0:
                s = jnp.concatenate(
                    [xb[:, sh:], jnp.zeros((C, sh), jnp.bfloat16)], axis=1)
            elif sh < 0:
                s = jnp.concatenate(
                    [jnp.zeros((C, -sh), jnp.bfloat16), xb[:, :sh]], axis=1)
            else:
                s = xb
            if dx == 1:
                s = s * mnr
            elif dx == -1:
                s = s * mnl
            acc += jnp.dot(w_ref[ib, t], s,
                           preferred_element_type=jnp.float32)
        o_ref[e] = jnp.maximum(acc, 0.0)


def _conv_call(idx, x3, CWr_bf):
    return pl.pallas_call(
        _conv_kernel,
        grid_spec=pltpu.PrefetchScalarGridSpec(
            num_scalar_prefetch=1,
            grid=(B // GB,),
            in_specs=[
                pl.BlockSpec((GB, C, HW), lambda g, idx_ref: (g, 0, 0)),
                pl.BlockSpec((U, 9, C, C),
                             lambda g, idx_ref: (0, 0, 0, 0)),
            ],
            out_specs=pl.BlockSpec((GB, C, HW), lambda g, idx_ref: (g, 0, 0)),
        ),
        out_shape=jax.ShapeDtypeStruct((B, C, HW), jnp.float32),
    )(idx, x3, CWr_bf)


def kernel(x, hash_W, bases, conv_W, depth):
    depth_arr = jnp.asarray(depth, jnp.int32).reshape(1)
    x3 = x.reshape(B, C, HW)
    B2bf = bases.transpose(1, 0, 2).reshape(D, UK).astype(jnp.bfloat16)
    B2Tbf = bases.transpose(0, 2, 1).reshape(UK, D).astype(jnp.bfloat16)
    CWr_bf = conv_W.transpose(0, 3, 4, 1, 2).reshape(U, 9, C, C).astype(
        jnp.bfloat16)

    outp = jnp.zeros((B, D), jnp.float32)
    avail = jnp.ones((B, U), jnp.float32)
    act = jnp.ones((B, 1), jnp.float32)
    routes = []
    xcur = x3
    hw_bf = None
    for d in range(DEPTH):
        if d == 0:
            (h, route_d, idx2, avail, outp, act,
             hw_bf) = _hash_route_call(
                d, depth_arr, xcur, hash_W, B2bf, B2Tbf, avail, outp, act,
                cast_hw=True)
        else:
            h, route_d, idx2, avail, outp, act = _hash_route_call(
                d, depth_arr, xcur, hw_bf, B2bf, B2Tbf, avail, outp, act)
        routes.append(route_d[:, 0])
        if d < DEPTH - 1:
            xcur = _conv_call(idx2[:, 0], xcur, CWr_bf)
    return outp, jnp.stack(routes, axis=1)


# speedup vs baseline: 10.3424x; 1.0008x over previous
import functools

import jax
import jax.numpy as jnp
from jax import lax
from jax.experimental import pallas as pl
from jax.experimental.pallas import tpu as pltpu

B = 64
U = 16
D = 256
K = 32
C = 32
HW = 1024
F = C * HW
DEPTH = 4
UK = U * K
UD = U * D
CBLK = 8
NKB = C // CBLK


def _hash_route_kernel(d_const, cast_hw, *refs):
    if cast_hw:
        (depth_ref, x_ref, hw_ref, b2_ref, b2t_ref,
         avail_ref, outp_ref, act_ref,
         h_ref, route_ref, idx_ref, avail_o_ref, out_o_ref, act_o_ref,
         hwbf_ref) = refs
    else:
        (depth_ref, x_ref, hw_ref, b2_ref, b2t_ref,
         avail_ref, outp_ref, act_ref,
         h_ref, route_ref, idx_ref, avail_o_ref, out_o_ref,
         act_o_ref) = refs
    kg = pl.program_id(0)

    @pl.when(kg == 0)
    def _init():
        h_ref[...] = jnp.zeros_like(h_ref)

    if cast_hw:
        hwc = hw_ref[...].astype(jnp.bfloat16)
        hwbf_ref[...] = hwc
    else:
        hwc = hw_ref[...]
    acc0 = jnp.zeros((B, D), jnp.float32)
    acc1 = jnp.zeros((B, D), jnp.float32)
    for j in range(CBLK):
        xc = x_ref[:, j, :].astype(jnp.bfloat16)
        p = jnp.dot(xc, hwc[j * HW:(j + 1) * HW, :],
                    preferred_element_type=jnp.float32)
        if j % 2 == 0:
            acc0 = acc0 + p
        else:
            acc1 = acc1 + p
    h_ref[...] += acc0 + acc1

    @pl.when(kg == NKB - 1)
    def _fin():
        h = h_ref[...]
        coef = jnp.dot(h.astype(jnp.bfloat16), b2_ref[...],
                       preferred_element_type=jnp.float32)
        cbf = coef.astype(jnp.bfloat16)
        projs = []
        m2 = []
        for u in range(U):
            pu = jnp.dot(cbf[:, u * K:(u + 1) * K],
                         b2t_ref[u * K:(u + 1) * K, :],
                         preferred_element_type=jnp.float32)
            projs.append(pu)
            m2.append(jnp.sum(pu * pu, axis=1))
        mags2 = jnp.stack(m2, axis=1)
        step_on = d_const < depth_ref[0]
        avail = avail_ref[...]
        masked = jnp.where(avail > 0.5, mags2, -jnp.inf)
        m = jnp.max(masked, axis=1, keepdims=True)
        iota = lax.broadcasted_iota(jnp.int32, (B, U), 1)
        idx = jnp.min(jnp.where(masked == m, iota, U + 1), axis=1)
        onehot = (iota == idx[:, None]).astype(jnp.float32)
        proj = jnp.zeros((B, D), jnp.float32)
        for u in range(U):
            proj = proj + projs[u] * onehot[:, u][:, None]
        residual = h - proj
        act = act_ref[...][:, 0]
        sonf = jnp.where(step_on, 1.0, 0.0)
        out_o_ref[...] = outp_ref[...] + residual * (act * sonf)[:, None]
        rnorm2 = jnp.sum(residual * residual, axis=1)
        live = (rnorm2 > 1e-10).astype(jnp.float32)
        act_o_ref[...] = jnp.where(step_on, act * live, act)[:, None]
        avail_o_ref[...] = jnp.where(step_on, avail * (1.0 - onehot), avail)
        route_ref[...] = jnp.where(step_on, idx, 0)[:, None]
        idx_ref[...] = idx[:, None]


def _hash_route_call(d, depth_arr, x3, hw, B2bf, B2Tbf, avail, outp, act,
                     cast_hw=False):
    full = lambda k: (0, 0)
    out_specs = [
        pl.BlockSpec((B, D), full),
        pl.BlockSpec((B, 1), full),
        pl.BlockSpec((B, 1), full),
        pl.BlockSpec((B, U), full),
        pl.BlockSpec((B, D), full),
        pl.BlockSpec((B, 1), full),
    ]
    out_shape = [
        jax.ShapeDtypeStruct((B, D), jnp.float32),
        jax.ShapeDtypeStruct((B, 1), jnp.int32),
        jax.ShapeDtypeStruct((B, 1), jnp.int32),
        jax.ShapeDtypeStruct((B, U), jnp.float32),
        jax.ShapeDtypeStruct((B, D), jnp.float32),
        jax.ShapeDtypeStruct((B, 1), jnp.float32),
    ]
    if cast_hw:
        out_specs.append(pl.BlockSpec((CBLK * HW, D), lambda k: (k, 0)))
        out_shape.append(jax.ShapeDtypeStruct((F, D), jnp.bfloat16))
    return pl.pallas_call(
        functools.partial(_hash_route_kernel, d, cast_hw),
        grid=(NKB,),
        in_specs=[
            pl.BlockSpec(memory_space=pltpu.SMEM),
            pl.BlockSpec((B, CBLK, HW), lambda k: (0, k, 0)),
            pl.BlockSpec((CBLK * HW, D), lambda k: (k, 0)),
            pl.BlockSpec((D, UK), full),
            pl.BlockSpec((UK, D), full),
            pl.BlockSpec((B, U), full),
            pl.BlockSpec((B, D), full),
            pl.BlockSpec((B, 1), full),
        ],
        out_specs=out_specs,
        out_shape=out_shape,
    )(depth_arr, x3, hw, B2bf, B2Tbf, avail, outp, act)


GB = 8


def _conv_kernel(idx_ref, x_ref, w_ref, o_ref):
    g = pl.program_id(0)
    wpos = lax.broadcasted_iota(jnp.int32, (C, HW), 1) % 32
    mnr = (wpos != 31).astype(jnp.bfloat16)
    mnl = (wpos != 0).astype(jnp.bfloat16)
    for e in range(GB):
        ib = idx_ref[g * GB + e]
        xb = x_ref[e].astype(jnp.bfloat16)
        acc = jnp.zeros((C, HW), jnp.float32)
        acc1 = jnp.zeros((C, HW), jnp.float32)
        for t in range(9):
            ky, kx = t // 3, t % 3
            dy, dx = ky - 1, kx - 1
            sh = 32 * dy + dx
            if sh > 0:
                s = jnp.concatenate(
                    [xb[:, sh:], jnp.zeros((C, sh), jnp.bfloat16)], axis=1)
            elif sh < 0:
                s = jnp.concatenate(
                    [jnp.zeros((C, -sh), jnp.bfloat16), xb[:, :sh]], axis=1)
            else:
                s = xb
            if dx == 1:
                s = s * mnr
            elif dx == -1:
                s = s * mnl
            p = jnp.dot(w_ref[ib, t], s,
                        preferred_element_type=jnp.float32)
            if t % 2 == 0:
                acc = acc + p
            else:
                acc1 = acc1 + p
        o_ref[e] = jnp.maximum(acc + acc1, 0.0)


def _conv_call(idx, x3, CWr_bf):
    return pl.pallas_call(
        _conv_kernel,
        grid_spec=pltpu.PrefetchScalarGridSpec(
            num_scalar_prefetch=1,
            grid=(B // GB,),
            in_specs=[
                pl.BlockSpec((GB, C, HW), lambda g, idx_ref: (g, 0, 0)),
                pl.BlockSpec((U, 9, C, C),
                             lambda g, idx_ref: (0, 0, 0, 0)),
            ],
            out_specs=pl.BlockSpec((GB, C, HW), lambda g, idx_ref: (g, 0, 0)),
        ),
        out_shape=jax.ShapeDtypeStruct((B, C, HW), jnp.float32),
    )(idx, x3, CWr_bf)


def kernel(x, hash_W, bases, conv_W, depth):
    depth_arr = jnp.asarray(depth, jnp.int32).reshape(1)
    x3 = x.reshape(B, C, HW)
    B2bf = bases.transpose(1, 0, 2).reshape(D, UK).astype(jnp.bfloat16)
    B2Tbf = bases.transpose(0, 2, 1).reshape(UK, D).astype(jnp.bfloat16)
    CWr_bf = conv_W.transpose(0, 3, 4, 1, 2).reshape(U, 9, C, C).astype(
        jnp.bfloat16)

    outp = jnp.zeros((B, D), jnp.float32)
    avail = jnp.ones((B, U), jnp.float32)
    act = jnp.ones((B, 1), jnp.float32)
    routes = []
    xcur = x3
    hw_bf = None
    for d in range(DEPTH):
        if d == 0:
            (h, route_d, idx2, avail, outp, act,
             hw_bf) = _hash_route_call(
                d, depth_arr, xcur, hash_W, B2bf, B2Tbf, avail, outp, act,
                cast_hw=True)
        else:
            h, route_d, idx2, avail, outp, act = _hash_route_call(
                d, depth_arr, xcur, hw_bf, B2bf, B2Tbf, avail, outp, act)
        routes.append(route_d[:, 0])
        if d < DEPTH - 1:
            xcur = _conv_call(idx2[:, 0], xcur, CWr_bf)
    return outp, jnp.stack(routes, axis=1)
